# baseline (speedup 1.0000x reference)
import functools

import jax
import jax.numpy as jnp
from jax import lax
from jax.experimental import pallas as pl
from jax.experimental.pallas import tpu as pltpu
from jax.experimental.pallas import tpu_sc as plsc

NUM_ENTITIES = 1000000
NUM_RELATIONS = 1000
EMBED_DIM = 64
BATCH = 16384

NC = 2
NS = 16
NW = NC * NS
ROWS_PER_W = BATCH // NW
CHUNK = 128
NCHUNK = ROWS_PER_W // CHUNK
LANES = 16
NVEC = EMBED_DIM // LANES

_GATHER_DNUMS = lax.GatherDimensionNumbers(
    offset_dims=(), collapsed_slice_dims=(0,), start_index_map=(0,))


def _permute(x, idx):
    return lax.gather(x, idx[:, None], _GATHER_DNUMS, (1,),
                      mode=lax.GatherScatterMode.PROMISE_IN_BOUNDS)


def _tec_body(heads_hbm, rels_hbm, ent_hbm, rel_hbm, out_hbm,
              idx_h, idx_r, hbuf, rbuf, sem):
    wid = lax.axis_index("c") * NS + lax.axis_index("s")
    base = wid * ROWS_PER_W

    pltpu.sync_copy(heads_hbm.at[pl.ds(wid * NCHUNK, NCHUNK)], idx_h)
    pltpu.sync_copy(rels_hbm.at[pl.ds(wid * NCHUNK, NCHUNK)], idx_r)

    descs = []
    for j in range(NCHUNK):
        descs.append(pltpu.async_copy(
            ent_hbm.at[idx_h.at[j]], hbuf.at[pl.ds(j * CHUNK, CHUNK)], sem))
        descs.append(pltpu.async_copy(
            rel_hbm.at[idx_r.at[j]], rbuf.at[pl.ds(j * CHUNK, CHUNK)], sem))
    for d in descs:
        d.wait()

    iota = lax.iota(jnp.int32, LANES)
    perms = [iota ^ sh for sh in (8, 4, 2, 1)]

    def row(i, _):
        vs = []
        ss = None
        for k in range(NVEC):
            v = hbuf[i, pl.ds(k * LANES, LANES)] + rbuf[i, pl.ds(k * LANES, LANES)]
            vs.append(v)
            sq = v * v
            ss = sq if ss is None else ss + sq
        for p in perms:
            ss = ss + _permute(ss, p)
        t = jnp.maximum(ss, jnp.float32(1e-24))
        bits = lax.bitcast_convert_type(t, jnp.int32)
        y = lax.bitcast_convert_type(
            jnp.int32(0x5F3759DF) - (bits >> 1), jnp.float32)
        for _ in range(3):
            y = y * (jnp.float32(1.5) - jnp.float32(0.5) * t * y * y)
        for k in range(NVEC):
            hbuf[i, pl.ds(k * LANES, LANES)] = vs[k] * y
        return 0

    lax.fori_loop(0, ROWS_PER_W, row, 0)

    pltpu.sync_copy(hbuf, out_hbm.at[pl.ds(base, ROWS_PER_W)])


@jax.jit
def _run(heads2, rels2, entity_table, relation_table):
    mesh = plsc.VectorSubcoreMesh(
        core_axis_name="c", subcore_axis_name="s",
        num_cores=NC, num_subcores=NS)
    return pl.kernel(
        _tec_body,
        out_type=jax.ShapeDtypeStruct((BATCH, EMBED_DIM), jnp.float32),
        mesh=mesh,
        scratch_types=[
            pltpu.VMEM((NCHUNK, CHUNK), jnp.int32),
            pltpu.VMEM((NCHUNK, CHUNK), jnp.int32),
            pltpu.VMEM((ROWS_PER_W, EMBED_DIM), jnp.float32),
            pltpu.VMEM((ROWS_PER_W, EMBED_DIM), jnp.float32),
            pltpu.SemaphoreType.DMA,
        ],
        compiler_params=pltpu.CompilerParams(use_tc_tiling_on_sc=False),
    )(heads2, rels2, entity_table, relation_table)


def kernel(heads, relations, entity_table, relation_table):
    heads2 = jnp.asarray(heads, jnp.int32).reshape(NW * NCHUNK, CHUNK)
    rels2 = jnp.asarray(relations, jnp.int32).reshape(NW * NCHUNK, CHUNK)
    return _run(heads2, rels2, entity_table, relation_table)


# speedup vs baseline: 1.6003x; 1.6003x over previous
import jax
import jax.numpy as jnp
from jax import lax
from jax.experimental import pallas as pl
from jax.experimental.pallas import tpu as pltpu
from jax.experimental.pallas import tpu_sc as plsc

NUM_ENTITIES = 1000000
NUM_RELATIONS = 1000
EMBED_DIM = 64
BATCH = 16384

NC = 2
NS = 16
NW = NC * NS
ROWS_PER_W = BATCH // NW
CHUNK = 64
NCHUNK = ROWS_PER_W // CHUNK
LANES = 16
NVEC = EMBED_DIM // LANES
GRP = 8
NGRP = NUM_ENTITIES // GRP

_GATHER_DNUMS = lax.GatherDimensionNumbers(
    offset_dims=(), collapsed_slice_dims=(0,), start_index_map=(0,))


def _permute(x, idx):
    return lax.gather(x, idx[:, None], _GATHER_DNUMS, (1,),
                      mode=lax.GatherScatterMode.PROMISE_IN_BOUNDS)


def _tec_body(heads_hbm, rels_hbm, ent_hbm, rel_hbm, out_hbm,
              hvec, rvec, reltab, entbuf, outbuf, sem):
    wid = lax.axis_index("c") * NS + lax.axis_index("s")
    base = wid * ROWS_PER_W

    pltpu.sync_copy(heads_hbm.at[pl.ds(base, ROWS_PER_W)],
                    hvec.at[pl.ds(0, ROWS_PER_W)])
    pltpu.sync_copy(rels_hbm.at[pl.ds(base, ROWS_PER_W)],
                    rvec.at[pl.ds(0, ROWS_PER_W)])
    pltpu.sync_copy(rel_hbm, reltab)

    iota = lax.iota(jnp.int32, LANES)
    perms = [iota ^ sh for sh in (8, 4, 2, 1)]

    def chunk(c, _):
        def fire(i, _):
            h = hvec[pl.ds(c * CHUNK + i, LANES)][0]
            pltpu.async_copy(
                ent_hbm.at[pl.ds(h, 1)], entbuf.at[pl.ds(i, 1)], sem)
            return 0

        lax.fori_loop(0, CHUNK, fire, 0)
        pltpu.make_async_copy(out_hbm.at[pl.ds(0, CHUNK)], entbuf, sem).wait()

        def row(i, _):
            r = rvec[pl.ds(c * CHUNK + i, LANES)][0]
            rq = r >> 1
            rp = (r & 1) * EMBED_DIM
            vs = []
            ss = None
            for k in range(NVEC):
                v = (entbuf[i, pl.ds(k * LANES, LANES)]
                     + reltab[rq, pl.ds(rp + k * LANES, LANES)])
                vs.append(v)
                sq = v * v
                ss = sq if ss is None else ss + sq
            for p in perms:
                ss = ss + _permute(ss, p)
            t = jnp.maximum(ss, jnp.float32(1e-24))
            bits = lax.bitcast_convert_type(t, jnp.int32)
            y = lax.bitcast_convert_type(
                jnp.int32(0x5F3759DF) - (bits >> 1), jnp.float32)
            for _ in range(3):
                y = y * (jnp.float32(1.5) - jnp.float32(0.5) * t * y * y)
            for k in range(NVEC):
                outbuf[i, pl.ds(k * LANES, LANES)] = vs[k] * y
            return 0

        lax.fori_loop(0, CHUNK, row, 0)
        pltpu.sync_copy(outbuf, out_hbm.at[pl.ds(base + c * CHUNK, CHUNK)])
        return 0

    lax.fori_loop(0, NCHUNK, chunk, 0)


@jax.jit
def _run(heads, rels, ent3, rel2):
    mesh = plsc.VectorSubcoreMesh(
        core_axis_name="c", subcore_axis_name="s",
        num_cores=NC, num_subcores=NS)
    return pl.kernel(
        _tec_body,
        out_type=jax.ShapeDtypeStruct((BATCH, EMBED_DIM), jnp.float32),
        mesh=mesh,
        scratch_types=[
            pltpu.VMEM((ROWS_PER_W + LANES,), jnp.int32),
            pltpu.VMEM((ROWS_PER_W + LANES,), jnp.int32),
            pltpu.VMEM((NUM_RELATIONS // 2, 2 * EMBED_DIM), jnp.float32),
            pltpu.VMEM((CHUNK, EMBED_DIM), jnp.float32),
            pltpu.VMEM((CHUNK, EMBED_DIM), jnp.float32),
            pltpu.SemaphoreType.DMA,
        ],
    )(heads, rels, ent3, rel2)


def kernel(heads, relations, entity_table, relation_table):
    heads = jnp.asarray(heads, jnp.int32)
    relations = jnp.asarray(relations, jnp.int32)
    rel2 = relation_table.reshape(NUM_RELATIONS // 2, 2 * EMBED_DIM)
    return _run(heads, relations, entity_table, rel2)
